# baseline (speedup 1.0000x reference)
import functools
import jax
import jax.numpy as jnp
from jax import lax
from jax.experimental import pallas as pl
from jax.experimental.pallas import tpu as pltpu
from jax.experimental.pallas import tpu_sc as plsc

N = 10000
Q = 5000
B = 16
DIN = 128
K = 16
H = 128
L = 256

N_PAD = 10240
Q_PAD = 5120
TQ = 256
TN = 1024
N_CHUNKS = N_PAD // TN
N_TILES = Q_PAD // TQ
AUGD = 256

_BIGV = 3e38
_BIGI = 1.0e9


def _sc_gather_body(n_rows, n_chunk, table_hbm, idx_hbm, out_hbm,
                    idx_v, rows_v, sem):
    info = plsc.get_sparse_core_info()
    nw = info.num_cores * info.num_subcores
    b_per_w = n_rows // nw
    wid = lax.axis_index("s") * info.num_cores + lax.axis_index("c")
    base = wid * b_per_w

    def step(g, _):
        off = base + g * n_chunk
        pltpu.sync_copy(idx_hbm.at[pl.ds(off, n_chunk)], idx_v)
        pltpu.async_copy(table_hbm.at[idx_v], rows_v, sem).wait()
        pltpu.sync_copy(rows_v, out_hbm.at[pl.ds(off, n_chunk)])
        return _

    lax.fori_loop(0, b_per_w // n_chunk, step, 0)


def _sc_gather(table, idx, n_chunk, out_dtype=jnp.float32):
    n_rows = idx.shape[0]
    d = table.shape[1]
    mesh = plsc.VectorSubcoreMesh(core_axis_name="c", subcore_axis_name="s")
    kfn = functools.partial(
        pl.kernel,
        mesh=mesh,
        out_type=jax.ShapeDtypeStruct((n_rows, d), out_dtype),
        scratch_types=[
            pltpu.VMEM((n_chunk,), jnp.int32),
            pltpu.VMEM((n_chunk, d), out_dtype),
            pltpu.SemaphoreType.DMA,
        ],
    )(functools.partial(_sc_gather_body, n_rows, n_chunk))
    return kfn(table, idx)


def _sc_scatter_body(n_rows, n_chunk, rows_hbm, idx_hbm, out_hbm,
                     idx_v, rows_v, sem):
    info = plsc.get_sparse_core_info()
    nw = info.num_cores * info.num_subcores
    b_per_w = n_rows // nw
    wid = lax.axis_index("s") * info.num_cores + lax.axis_index("c")
    base = wid * b_per_w

    def step(g, _):
        off = base + g * n_chunk
        pltpu.sync_copy(idx_hbm.at[pl.ds(off, n_chunk)], idx_v)
        pltpu.sync_copy(rows_hbm.at[pl.ds(off, n_chunk)], rows_v)
        pltpu.async_copy(rows_v, out_hbm.at[idx_v], sem).wait()
        return _

    lax.fori_loop(0, b_per_w // n_chunk, step, 0)


def _sc_scatter(rows, idx, n_chunk):
    n_rows, d = rows.shape
    mesh = plsc.VectorSubcoreMesh(core_axis_name="c", subcore_axis_name="s")
    kfn = functools.partial(
        pl.kernel,
        mesh=mesh,
        out_type=jax.ShapeDtypeStruct((n_rows, d), rows.dtype),
        scratch_types=[
            pltpu.VMEM((n_chunk,), jnp.int32),
            pltpu.VMEM((n_chunk, d), rows.dtype),
            pltpu.SemaphoreType.DMA,
        ],
    )(functools.partial(_sc_scatter_body, n_rows, n_chunk))
    return kfn(rows, idx)


def _rank_body(qaug_ref, qbrow_ref, bcol_ref, rank_ref, segs_ref):
    i = pl.program_id(0)

    @pl.when(i == 0)
    def _():
        bio16 = lax.broadcasted_iota(jnp.int32, (1, B), 1).astype(jnp.float32)
        onehot_nb = (bcol_ref[:, 0:1] == bio16).astype(jnp.float32)
        cnt = jnp.sum(onehot_nb, axis=0, keepdims=True)
        r16 = lax.broadcasted_iota(jnp.int32, (B, B), 0)
        c16 = lax.broadcasted_iota(jnp.int32, (B, B), 1)
        ltb = (r16 < c16).astype(jnp.float32)
        cum = jnp.dot(cnt, ltb, preferred_element_type=jnp.float32)
        pad = jnp.zeros((1, 128 - B), jnp.float32)
        cnt128 = jnp.concatenate([cnt, pad], axis=1)
        cum128 = jnp.concatenate([cum, pad], axis=1)
        rio = lax.broadcasted_iota(jnp.int32, (8, 128), 0)
        segs_ref[...] = jnp.where(rio == 0,
                                  jnp.broadcast_to(cnt128, (8, 128)),
                                  jnp.where(rio == 1,
                                            jnp.broadcast_to(cum128, (8, 128)),
                                            0.0))

    qb = qaug_ref[:, DIN + 3:DIN + 4]
    qrow = qbrow_ref[0:1, :]
    less = jnp.sum((qrow < qb).astype(jnp.float32), axis=1, keepdims=True)
    colidx = lax.broadcasted_iota(jnp.int32, (1, Q_PAD), 1)
    before = colidx < i * TQ
    eq_before = jnp.sum(
        jnp.where(jnp.logical_and(qrow == qb, before), 1.0, 0.0),
        axis=1, keepdims=True)
    rr = lax.broadcasted_iota(jnp.int32, (TQ, TQ), 0)
    cc = lax.broadcasted_iota(jnp.int32, (TQ, TQ), 1)
    ltq = (cc < rr).astype(jnp.float32)
    bio = lax.broadcasted_iota(jnp.int32, (1, B), 1).astype(jnp.float32)
    onehot = (qb == bio).astype(jnp.float32)
    pref = jnp.dot(ltq, onehot, preferred_element_type=jnp.float32)
    eq_tile = jnp.sum(onehot * pref, axis=1, keepdims=True)
    rank = less + eq_before + eq_tile
    rank_ref[...] = rank.astype(jnp.int32)


def _rank(qaug, qbrow, bcol):
    return pl.pallas_call(
        _rank_body,
        grid=(N_TILES,),
        in_specs=[
            pl.BlockSpec((TQ, AUGD), lambda i: (i, 0)),
            pl.BlockSpec((8, Q_PAD), lambda i: (0, 0)),
            pl.BlockSpec((N_PAD, 8), lambda i: (0, 0)),
        ],
        out_specs=[
            pl.BlockSpec((TQ, 1), lambda i: (i, 0)),
            pl.BlockSpec((8, 128), lambda i: (0, 0)),
        ],
        out_shape=[
            jax.ShapeDtypeStruct((Q_PAD, 1), jnp.int32),
            jax.ShapeDtypeStruct((8, 128), jnp.float32),
        ],
    )(qaug, qbrow, bcol)


def _knn_body(qaug_ref, xyzt_ref, brow_ref, segs_ref, idx_ref):
    qx = qaug_ref[:, DIN:DIN + 1]
    qy = qaug_ref[:, DIN + 1:DIN + 2]
    qz = qaug_ref[:, DIN + 2:DIN + 3]
    qb = qaug_ref[:, DIN + 3:DIN + 4]

    kcol = lax.broadcasted_iota(jnp.int32, (1, K), 1)
    brow = brow_ref[0:1, :]

    bmin = jnp.min(qb)
    bmax = jnp.max(qb)
    bio16 = lax.broadcasted_iota(jnp.int32, (1, B), 1).astype(jnp.float32)
    cnt = segs_ref[0:1, 0:B]
    cum = segs_ref[1:2, 0:B]
    inb = jnp.logical_and(bio16 >= bmin, bio16 <= bmax)
    minlen = jnp.min(jnp.where(inb, cnt, _BIGV))
    start = jnp.sum(jnp.where(bio16 == bmin, cum, 0.0)).astype(jnp.int32)
    end = jnp.sum(jnp.where(bio16 == bmax, cum + cnt, 0.0)).astype(jnp.int32)
    narrow = minlen >= jnp.float32(K)
    c0 = jnp.where(narrow, start // TN, 0)
    c1 = jnp.where(narrow, (end + TN - 1) // TN, N_CHUNKS)

    def chunk_body(c, carry):
        bestv, besti = carry
        off = c * TN
        sx = xyzt_ref[0:1, pl.ds(off, TN)]
        sy = xyzt_ref[1:2, pl.ds(off, TN)]
        sz = xyzt_ref[2:3, pl.ds(off, TN)]
        sb = brow_ref[0:1, pl.ds(off, TN)]
        dx = qx - sx
        dy = qy - sy
        dz = qz - sz
        d2 = dx * dx + dy * dy
        d2 = d2 + dz * dz
        d2 = d2 + jnp.where(qb != sb, 1e9, 0.0)
        gidx = (off + lax.broadcasted_iota(jnp.int32, (1, TN), 1)).astype(
            jnp.float32)

        hw = TN // 2
        a = d2[:, :hw]
        b2 = d2[:, hw:]
        ia = jnp.broadcast_to(gidx[:, :hw], (TQ, hw))
        ib = jnp.broadcast_to(gidx[:, hw:], (TQ, hw))
        cond = a <= b2
        pm = jnp.where(cond, a, b2)
        px = jnp.where(cond, b2, a)
        pi_mn = jnp.where(cond, ia, ib)
        pi_mx = jnp.where(cond, ib, ia)

        kv = []
        ki = []
        for k in range(K):
            m = jnp.min(pm, axis=1, keepdims=True)
            ji = jnp.min(jnp.where(pm == m, pi_mn, _BIGI), axis=1,
                         keepdims=True)
            sel = pi_mn == ji
            pm = jnp.where(sel, px, pm)
            pi_mn = jnp.where(sel, pi_mx, pi_mn)
            px = jnp.where(sel, _BIGV, px)
            kv.append(m)
            ki.append(ji)
        chv = jnp.concatenate(kv, axis=1)
        chi = jnp.concatenate(ki, axis=1)

        wv = jnp.concatenate([bestv, chv], axis=1)
        wi = jnp.concatenate([besti, chi], axis=1)
        bv = bestv
        bi = besti
        for k in range(K):
            m = jnp.min(wv, axis=1, keepdims=True)
            ji = jnp.min(jnp.where(wv == m, wi, _BIGI), axis=1, keepdims=True)
            wv = jnp.where(wi == ji, _BIGV, wv)
            sel = kcol == k
            bv = jnp.where(sel, m, bv)
            bi = jnp.where(sel, ji, bi)
        return bv, bi

    bestv0 = jnp.full((TQ, K), _BIGV, jnp.float32)
    besti0 = jnp.full((TQ, K), -1.0, jnp.float32)
    _, besti = lax.fori_loop(c0, c1, chunk_body, (bestv0, besti0))
    idx_ref[...] = besti.astype(jnp.int32)


def _knn(qaug, xyzt, brow, segs):
    nq = qaug.shape[0]
    return pl.pallas_call(
        _knn_body,
        grid=(nq // TQ,),
        in_specs=[
            pl.BlockSpec((TQ, AUGD), lambda i: (i, 0)),
            pl.BlockSpec((8, N_PAD), lambda i: (0, 0)),
            pl.BlockSpec((8, N_PAD), lambda i: (0, 0)),
            pl.BlockSpec((8, 128), lambda i: (0, 0)),
        ],
        out_specs=pl.BlockSpec((TQ, K), lambda i: (i, 0)),
        out_shape=jax.ShapeDtypeStruct((nq, K), jnp.int32),
    )(qaug, xyzt, brow, segs)


def _head_body(qaug_ref, xj_ref, lang_ref, wl1_ref, bl1_ref, bng_ref, bnb_ref,
               wl2_ref, bl2_ref, we_ref, be_ref, wv1_ref, bv1_ref, lng_ref,
               lnb_ref, wv2_ref, bv2_ref, out_ref):
    l = jnp.dot(lang_ref[...], wl1_ref[...],
                preferred_element_type=jnp.float32) + bl1_ref[...]
    l = l / jnp.sqrt(1.0 + 1e-5) * bng_ref[...] + bnb_ref[...]
    l = jnp.maximum(l, 0.0)
    l = jnp.dot(l, wl2_ref[...],
                preferred_element_type=jnp.float32) + bl2_ref[...]

    x_i = qaug_ref[:, 0:DIN]
    qb = qaug_ref[:, DIN + 3:DIN + 4]

    we = we_ref[...]
    be = be_ref[...]
    g = jnp.full((TQ, H), -_BIGV, jnp.float32)
    for k in range(K):
        xj = xj_ref[k]
        e = jnp.concatenate([x_i, xj - x_i], axis=1)
        h = jnp.dot(e, we, preferred_element_type=jnp.float32) + be
        g = jnp.maximum(g, jnp.maximum(h, 0.0))

    v = jnp.dot(g, wv1_ref[...],
                preferred_element_type=jnp.float32) + bv1_ref[...]
    mu = jnp.mean(v, axis=1, keepdims=True)
    var = jnp.mean((v - mu) * (v - mu), axis=1, keepdims=True)
    v = (v - mu) / jnp.sqrt(var + 1e-5) * lng_ref[...] + lnb_ref[...]
    v = jnp.maximum(v, 0.0)
    v = jnp.dot(v, wv2_ref[...],
                preferred_element_type=jnp.float32) + bv2_ref[...]

    bio = lax.broadcasted_iota(jnp.int32, (1, B), 1).astype(jnp.float32)
    onehot = (qb == bio).astype(jnp.float32)
    lq = jnp.dot(onehot, l, preferred_element_type=jnp.float32)

    num = jnp.sum(v * lq, axis=1, keepdims=True)
    nv = jnp.sqrt(jnp.sum(v * v, axis=1, keepdims=True))
    nl = jnp.sqrt(jnp.sum(lq * lq, axis=1, keepdims=True))
    den = jnp.maximum(nv * nl, 1e-8)
    out_ref[...] = jnp.broadcast_to(num / den, (TQ, 128))


def _head(qaug, xj3, lang, wl1, bl1, bng, bnb, wl2, bl2, we, be, wv1, bv1,
          lng, lnb, wv2, bv2):
    nq = qaug.shape[0]
    full = lambda shape: pl.BlockSpec(shape, lambda i: tuple(0 for _ in shape))
    return pl.pallas_call(
        _head_body,
        grid=(nq // TQ,),
        in_specs=[
            pl.BlockSpec((TQ, AUGD), lambda i: (i, 0)),
            pl.BlockSpec((K, TQ, DIN), lambda i: (0, i, 0)),
            full((B, L)),
            full((L, H)), full((1, H)), full((1, H)), full((1, H)),
            full((H, H)), full((1, H)),
            full((2 * DIN, H)), full((1, H)),
            full((H, H)), full((1, H)), full((1, H)), full((1, H)),
            full((H, H)), full((1, H)),
        ],
        out_specs=pl.BlockSpec((TQ, 128), lambda i: (i, 0)),
        out_shape=jax.ShapeDtypeStruct((nq, 128), jnp.float32),
    )(qaug, xj3, lang, wl1, bl1, bng, bnb, wl2, bl2, we, be, wv1, bv1,
      lng, lnb, wv2, bv2)


def kernel(support_xyz, batch_index, filtered_index, feats, lang_rel_feats,
           W_l1, b_l1, bn_g, bn_b, W_l2, b_l2,
           W_e, b_e, W_v1, b_v1, ln_g, ln_b, W_v2, b_v2):
    batch_index = batch_index.astype(jnp.int32)
    filtered_index = filtered_index.astype(jnp.int32)

    aug = jnp.concatenate([
        feats,
        support_xyz,
        batch_index[:, None].astype(jnp.float32),
        jnp.zeros((N, AUGD - DIN - 4), jnp.float32),
    ], axis=1)

    fi_pad = jnp.concatenate(
        [filtered_index, jnp.zeros((Q_PAD - Q,), jnp.int32)])

    qaug = _sc_gather(aug, fi_pad, 160)

    qbrow = jnp.broadcast_to(qaug[:, DIN + 3:DIN + 4].T, (8, Q_PAD))
    bcol = jnp.full((N_PAD, 8), 99.0, jnp.float32)
    bcol = bcol.at[0:N, 0].set(batch_index.astype(jnp.float32))
    rank, segs = _rank(qaug, qbrow, bcol)
    rank_flat = rank.reshape(-1)
    qaug_s = _sc_scatter(qaug, rank_flat, 80)

    xyzt = jnp.full((8, N_PAD), 1e5, jnp.float32)
    xyzt = xyzt.at[0:3, 0:N].set(support_xyz.T)
    brow = jnp.full((8, N_PAD), 99.0, jnp.float32)
    brow = brow.at[0, 0:N].set(batch_index.astype(jnp.float32))

    half = Q_PAD // 2
    scores_halves = []
    for qh in (qaug_s[:half], qaug_s[half:]):
        idx_h = _knn(qh, xyzt, brow, segs)
        idxf_h = idx_h.T.reshape(-1)
        xj_h = _sc_gather(feats, idxf_h, 320)
        xj3_h = xj_h.reshape(K, half, DIN)
        scores_halves.append(
            _head(qh, xj3_h, lang_rel_feats, W_l1, b_l1[None, :],
                  bn_g[None, :], bn_b[None, :], W_l2, b_l2[None, :],
                  W_e, b_e[None, :], W_v1, b_v1[None, :], ln_g[None, :],
                  ln_b[None, :], W_v2, b_v2[None, :]))
    scores_t = jnp.concatenate(scores_halves, axis=0)

    scores = _sc_gather(scores_t, rank_flat, 160)
    return scores[:Q, 0]


# speedup vs baseline: 1.2054x; 1.2054x over previous
import functools
import jax
import jax.numpy as jnp
from jax import lax
from jax.experimental import pallas as pl
from jax.experimental.pallas import tpu as pltpu
from jax.experimental.pallas import tpu_sc as plsc

N = 10000
Q = 5000
B = 16
DIN = 128
K = 16
H = 128
L = 256

N_PAD = 10240
Q_PAD = 5120
TQ = 256
TN = 1024
N_CHUNKS = N_PAD // TN
N_TILES = Q_PAD // TQ
AUGD = 256

_BIGV = 3e38
_BIGI = 1.0e9


def _sc_gather_body(n_rows, n_chunk, table_hbm, idx_hbm, out_hbm,
                    idx_v, rows_v, sem):
    info = plsc.get_sparse_core_info()
    nw = info.num_cores * info.num_subcores
    b_per_w = n_rows // nw
    wid = lax.axis_index("s") * info.num_cores + lax.axis_index("c")
    base = wid * b_per_w

    def step(g, _):
        off = base + g * n_chunk
        pltpu.sync_copy(idx_hbm.at[pl.ds(off, n_chunk)], idx_v)
        pltpu.async_copy(table_hbm.at[idx_v], rows_v, sem).wait()
        pltpu.sync_copy(rows_v, out_hbm.at[pl.ds(off, n_chunk)])
        return _

    lax.fori_loop(0, b_per_w // n_chunk, step, 0)


def _sc_gather(table, idx, n_chunk, out_dtype=jnp.float32):
    n_rows = idx.shape[0]
    d = table.shape[1]
    mesh = plsc.VectorSubcoreMesh(core_axis_name="c", subcore_axis_name="s")
    kfn = functools.partial(
        pl.kernel,
        mesh=mesh,
        out_type=jax.ShapeDtypeStruct((n_rows, d), out_dtype),
        scratch_types=[
            pltpu.VMEM((n_chunk,), jnp.int32),
            pltpu.VMEM((n_chunk, d), out_dtype),
            pltpu.SemaphoreType.DMA,
        ],
    )(functools.partial(_sc_gather_body, n_rows, n_chunk))
    return kfn(table, idx)


def _sc_scatter_body(n_rows, n_chunk, rows_hbm, idx_hbm, out_hbm,
                     idx_v, rows_v, sem):
    info = plsc.get_sparse_core_info()
    nw = info.num_cores * info.num_subcores
    b_per_w = n_rows // nw
    wid = lax.axis_index("s") * info.num_cores + lax.axis_index("c")
    base = wid * b_per_w

    def step(g, _):
        off = base + g * n_chunk
        pltpu.sync_copy(idx_hbm.at[pl.ds(off, n_chunk)], idx_v)
        pltpu.sync_copy(rows_hbm.at[pl.ds(off, n_chunk)], rows_v)
        pltpu.async_copy(rows_v, out_hbm.at[idx_v], sem).wait()
        return _

    lax.fori_loop(0, b_per_w // n_chunk, step, 0)


def _sc_scatter(rows, idx, n_chunk):
    n_rows, d = rows.shape
    mesh = plsc.VectorSubcoreMesh(core_axis_name="c", subcore_axis_name="s")
    kfn = functools.partial(
        pl.kernel,
        mesh=mesh,
        out_type=jax.ShapeDtypeStruct((n_rows, d), rows.dtype),
        scratch_types=[
            pltpu.VMEM((n_chunk,), jnp.int32),
            pltpu.VMEM((n_chunk, d), rows.dtype),
            pltpu.SemaphoreType.DMA,
        ],
    )(functools.partial(_sc_scatter_body, n_rows, n_chunk))
    return kfn(rows, idx)


def _rank_body(qaug_ref, qbrow_ref, bcol_ref, rank_ref, segs_ref):
    i = pl.program_id(0)

    @pl.when(i == 0)
    def _():
        bio16 = lax.broadcasted_iota(jnp.int32, (1, B), 1).astype(jnp.float32)
        onehot_nb = (bcol_ref[:, 0:1] == bio16).astype(jnp.float32)
        cnt = jnp.sum(onehot_nb, axis=0, keepdims=True)
        r16 = lax.broadcasted_iota(jnp.int32, (B, B), 0)
        c16 = lax.broadcasted_iota(jnp.int32, (B, B), 1)
        ltb = (r16 < c16).astype(jnp.float32)
        cum = jnp.dot(cnt, ltb, preferred_element_type=jnp.float32)
        pad = jnp.zeros((1, 128 - B), jnp.float32)
        cnt128 = jnp.concatenate([cnt, pad], axis=1)
        cum128 = jnp.concatenate([cum, pad], axis=1)
        rio = lax.broadcasted_iota(jnp.int32, (8, 128), 0)
        segs_ref[...] = jnp.where(rio == 0,
                                  jnp.broadcast_to(cnt128, (8, 128)),
                                  jnp.where(rio == 1,
                                            jnp.broadcast_to(cum128, (8, 128)),
                                            0.0))

    qb = qaug_ref[:, DIN + 3:DIN + 4]
    qrow = qbrow_ref[0:1, :]
    less = jnp.sum((qrow < qb).astype(jnp.float32), axis=1, keepdims=True)
    colidx = lax.broadcasted_iota(jnp.int32, (1, Q_PAD), 1)
    before = colidx < i * TQ
    eq_before = jnp.sum(
        jnp.where(jnp.logical_and(qrow == qb, before), 1.0, 0.0),
        axis=1, keepdims=True)
    rr = lax.broadcasted_iota(jnp.int32, (TQ, TQ), 0)
    cc = lax.broadcasted_iota(jnp.int32, (TQ, TQ), 1)
    ltq = (cc < rr).astype(jnp.float32)
    bio = lax.broadcasted_iota(jnp.int32, (1, B), 1).astype(jnp.float32)
    onehot = (qb == bio).astype(jnp.float32)
    pref = jnp.dot(ltq, onehot, preferred_element_type=jnp.float32)
    eq_tile = jnp.sum(onehot * pref, axis=1, keepdims=True)
    rank = less + eq_before + eq_tile
    rank_ref[...] = rank.astype(jnp.int32)


def _rank(qaug, qbrow, bcol):
    return pl.pallas_call(
        _rank_body,
        grid=(N_TILES,),
        in_specs=[
            pl.BlockSpec((TQ, AUGD), lambda i: (i, 0)),
            pl.BlockSpec((8, Q_PAD), lambda i: (0, 0)),
            pl.BlockSpec((N_PAD, 8), lambda i: (0, 0)),
        ],
        out_specs=[
            pl.BlockSpec((TQ, 1), lambda i: (i, 0)),
            pl.BlockSpec((8, 128), lambda i: (0, 0)),
        ],
        out_shape=[
            jax.ShapeDtypeStruct((Q_PAD, 1), jnp.int32),
            jax.ShapeDtypeStruct((8, 128), jnp.float32),
        ],
    )(qaug, qbrow, bcol)


def _knn_body(qaug_ref, xyzt_ref, brow_ref, segs_ref, idx_ref):
    qx = qaug_ref[:, DIN:DIN + 1]
    qy = qaug_ref[:, DIN + 1:DIN + 2]
    qz = qaug_ref[:, DIN + 2:DIN + 3]
    qb = qaug_ref[:, DIN + 3:DIN + 4]

    kcol = lax.broadcasted_iota(jnp.int32, (1, K), 1)
    brow = brow_ref[0:1, :]

    bmin = jnp.min(qb)
    bmax = jnp.max(qb)
    bio16 = lax.broadcasted_iota(jnp.int32, (1, B), 1).astype(jnp.float32)
    cnt = segs_ref[0:1, 0:B]
    cum = segs_ref[1:2, 0:B]
    inb = jnp.logical_and(bio16 >= bmin, bio16 <= bmax)
    minlen = jnp.min(jnp.where(inb, cnt, _BIGV))
    start = jnp.sum(jnp.where(bio16 == bmin, cum, 0.0)).astype(jnp.int32)
    end = jnp.sum(jnp.where(bio16 == bmax, cum + cnt, 0.0)).astype(jnp.int32)
    narrow = minlen >= jnp.float32(K)
    c0 = jnp.where(narrow, start // TN, 0)
    c1 = jnp.where(narrow, (end + TN - 1) // TN, N_CHUNKS)

    def chunk_body(c, carry):
        bestv, besti = carry
        off = c * TN
        sx = xyzt_ref[0:1, pl.ds(off, TN)]
        sy = xyzt_ref[1:2, pl.ds(off, TN)]
        sz = xyzt_ref[2:3, pl.ds(off, TN)]
        sb = brow_ref[0:1, pl.ds(off, TN)]
        dx = qx - sx
        dy = qy - sy
        dz = qz - sz
        d2 = dx * dx + dy * dy
        d2 = d2 + dz * dz
        d2 = d2 + jnp.where(qb != sb, 1e9, 0.0)
        gidx = (off + lax.broadcasted_iota(jnp.int32, (1, TN), 1)).astype(
            jnp.float32)
        wv = jnp.concatenate([bestv, d2], axis=1)
        wi = jnp.concatenate([besti, jnp.broadcast_to(gidx, (TQ, TN))], axis=1)

        bv = jnp.full((TQ, K), _BIGV, jnp.float32)
        bi = jnp.full((TQ, K), -1.0, jnp.float32)
        for k in range(K):
            m = jnp.min(wv, axis=1, keepdims=True)
            ji = jnp.min(jnp.where(wv == m, wi, _BIGI), axis=1, keepdims=True)
            wv = jnp.where(wi == ji, _BIGV, wv)
            sel = kcol == k
            bv = jnp.where(sel, m, bv)
            bi = jnp.where(sel, ji, bi)
        return bv, bi

    bestv0 = jnp.full((TQ, K), _BIGV, jnp.float32)
    besti0 = jnp.full((TQ, K), -1.0, jnp.float32)
    _, besti = lax.fori_loop(c0, c1, chunk_body, (bestv0, besti0))
    idx_ref[...] = besti.astype(jnp.int32)


def _knn(qaug, xyzt, brow, segs):
    nq = qaug.shape[0]
    return pl.pallas_call(
        _knn_body,
        grid=(nq // TQ,),
        in_specs=[
            pl.BlockSpec((TQ, AUGD), lambda i: (i, 0)),
            pl.BlockSpec((8, N_PAD), lambda i: (0, 0)),
            pl.BlockSpec((8, N_PAD), lambda i: (0, 0)),
            pl.BlockSpec((8, 128), lambda i: (0, 0)),
        ],
        out_specs=pl.BlockSpec((TQ, K), lambda i: (i, 0)),
        out_shape=jax.ShapeDtypeStruct((nq, K), jnp.int32),
    )(qaug, xyzt, brow, segs)


def _head_body(qaug_ref, xj_ref, lang_ref, wl1_ref, bl1_ref, bng_ref, bnb_ref,
               wl2_ref, bl2_ref, we_ref, be_ref, wv1_ref, bv1_ref, lng_ref,
               lnb_ref, wv2_ref, bv2_ref, out_ref):
    l = jnp.dot(lang_ref[...], wl1_ref[...],
                preferred_element_type=jnp.float32) + bl1_ref[...]
    l = l / jnp.sqrt(1.0 + 1e-5) * bng_ref[...] + bnb_ref[...]
    l = jnp.maximum(l, 0.0)
    l = jnp.dot(l, wl2_ref[...],
                preferred_element_type=jnp.float32) + bl2_ref[...]

    x_i = qaug_ref[:, 0:DIN]
    qb = qaug_ref[:, DIN + 3:DIN + 4]

    we = we_ref[...]
    be = be_ref[...]
    g = jnp.full((TQ, H), -_BIGV, jnp.float32)
    for k in range(K):
        xj = xj_ref[k]
        e = jnp.concatenate([x_i, xj - x_i], axis=1)
        h = jnp.dot(e, we, preferred_element_type=jnp.float32) + be
        g = jnp.maximum(g, jnp.maximum(h, 0.0))

    v = jnp.dot(g, wv1_ref[...],
                preferred_element_type=jnp.float32) + bv1_ref[...]
    mu = jnp.mean(v, axis=1, keepdims=True)
    var = jnp.mean((v - mu) * (v - mu), axis=1, keepdims=True)
    v = (v - mu) / jnp.sqrt(var + 1e-5) * lng_ref[...] + lnb_ref[...]
    v = jnp.maximum(v, 0.0)
    v = jnp.dot(v, wv2_ref[...],
                preferred_element_type=jnp.float32) + bv2_ref[...]

    bio = lax.broadcasted_iota(jnp.int32, (1, B), 1).astype(jnp.float32)
    onehot = (qb == bio).astype(jnp.float32)
    lq = jnp.dot(onehot, l, preferred_element_type=jnp.float32)

    num = jnp.sum(v * lq, axis=1, keepdims=True)
    nv = jnp.sqrt(jnp.sum(v * v, axis=1, keepdims=True))
    nl = jnp.sqrt(jnp.sum(lq * lq, axis=1, keepdims=True))
    den = jnp.maximum(nv * nl, 1e-8)
    out_ref[...] = jnp.broadcast_to(num / den, (TQ, 128))


def _head(qaug, xj3, lang, wl1, bl1, bng, bnb, wl2, bl2, we, be, wv1, bv1,
          lng, lnb, wv2, bv2):
    nq = qaug.shape[0]
    full = lambda shape: pl.BlockSpec(shape, lambda i: tuple(0 for _ in shape))
    return pl.pallas_call(
        _head_body,
        grid=(nq // TQ,),
        in_specs=[
            pl.BlockSpec((TQ, AUGD), lambda i: (i, 0)),
            pl.BlockSpec((K, TQ, DIN), lambda i: (0, i, 0)),
            full((B, L)),
            full((L, H)), full((1, H)), full((1, H)), full((1, H)),
            full((H, H)), full((1, H)),
            full((2 * DIN, H)), full((1, H)),
            full((H, H)), full((1, H)), full((1, H)), full((1, H)),
            full((H, H)), full((1, H)),
        ],
        out_specs=pl.BlockSpec((TQ, 128), lambda i: (i, 0)),
        out_shape=jax.ShapeDtypeStruct((nq, 128), jnp.float32),
    )(qaug, xj3, lang, wl1, bl1, bng, bnb, wl2, bl2, we, be, wv1, bv1,
      lng, lnb, wv2, bv2)


def kernel(support_xyz, batch_index, filtered_index, feats, lang_rel_feats,
           W_l1, b_l1, bn_g, bn_b, W_l2, b_l2,
           W_e, b_e, W_v1, b_v1, ln_g, ln_b, W_v2, b_v2):
    batch_index = batch_index.astype(jnp.int32)
    filtered_index = filtered_index.astype(jnp.int32)

    aug = jnp.concatenate([
        feats,
        support_xyz,
        batch_index[:, None].astype(jnp.float32),
        jnp.zeros((N, AUGD - DIN - 4), jnp.float32),
    ], axis=1)

    fi_pad = jnp.concatenate(
        [filtered_index, jnp.zeros((Q_PAD - Q,), jnp.int32)])

    qaug = _sc_gather(aug, fi_pad, 160)

    qbrow = jnp.broadcast_to(qaug[:, DIN + 3:DIN + 4].T, (8, Q_PAD))
    bcol = jnp.full((N_PAD, 8), 99.0, jnp.float32)
    bcol = bcol.at[0:N, 0].set(batch_index.astype(jnp.float32))
    rank, segs = _rank(qaug, qbrow, bcol)
    rank_flat = rank.reshape(-1)
    qaug_s = _sc_scatter(qaug, rank_flat, 80)

    xyzt = jnp.full((8, N_PAD), 1e5, jnp.float32)
    xyzt = xyzt.at[0:3, 0:N].set(support_xyz.T)
    brow = jnp.full((8, N_PAD), 99.0, jnp.float32)
    brow = brow.at[0, 0:N].set(batch_index.astype(jnp.float32))

    half = Q_PAD // 2
    scores_halves = []
    for qh in (qaug_s[:half], qaug_s[half:]):
        idx_h = _knn(qh, xyzt, brow, segs)
        idxf_h = idx_h.T.reshape(-1)
        xj_h = _sc_gather(feats, idxf_h, 320)
        xj3_h = xj_h.reshape(K, half, DIN)
        scores_halves.append(
            _head(qh, xj3_h, lang_rel_feats, W_l1, b_l1[None, :],
                  bn_g[None, :], bn_b[None, :], W_l2, b_l2[None, :],
                  W_e, b_e[None, :], W_v1, b_v1[None, :], ln_g[None, :],
                  ln_b[None, :], W_v2, b_v2[None, :]))
    scores_t = jnp.concatenate(scores_halves, axis=0)

    scores = _sc_gather(scores_t, rank_flat, 160)
    return scores[:Q, 0]


# speedup vs baseline: 1.2275x; 1.0183x over previous
import functools
import jax
import jax.numpy as jnp
from jax import lax
from jax.experimental import pallas as pl
from jax.experimental.pallas import tpu as pltpu
from jax.experimental.pallas import tpu_sc as plsc

N = 10000
Q = 5000
B = 16
DIN = 128
K = 16
H = 128
L = 256

N_PAD = 10240
Q_PAD = 5120
TQ = 256
TN = 1024
N_CHUNKS = N_PAD // TN
N_TILES = Q_PAD // TQ
AUGD = 256

_BIGV = 3e38
_BIGI = 1.0e9


def _sc_gather_body(n_rows, n_chunk, table_hbm, idx_hbm, out_hbm,
                    idx_v, rows_v, sem):
    info = plsc.get_sparse_core_info()
    nw = info.num_cores * info.num_subcores
    b_per_w = n_rows // nw
    wid = lax.axis_index("s") * info.num_cores + lax.axis_index("c")
    base = wid * b_per_w

    def step(g, _):
        off = base + g * n_chunk
        pltpu.sync_copy(idx_hbm.at[pl.ds(off, n_chunk)], idx_v)
        pltpu.async_copy(table_hbm.at[idx_v], rows_v, sem).wait()
        pltpu.sync_copy(rows_v, out_hbm.at[pl.ds(off, n_chunk)])
        return _

    lax.fori_loop(0, b_per_w // n_chunk, step, 0)


def _sc_gather(table, idx, n_chunk, out_dtype=jnp.float32):
    n_rows = idx.shape[0]
    d = table.shape[1]
    mesh = plsc.VectorSubcoreMesh(core_axis_name="c", subcore_axis_name="s")
    kfn = functools.partial(
        pl.kernel,
        mesh=mesh,
        out_type=jax.ShapeDtypeStruct((n_rows, d), out_dtype),
        scratch_types=[
            pltpu.VMEM((n_chunk,), jnp.int32),
            pltpu.VMEM((n_chunk, d), out_dtype),
            pltpu.SemaphoreType.DMA,
        ],
    )(functools.partial(_sc_gather_body, n_rows, n_chunk))
    return kfn(table, idx)


def _sc_scatter_body(n_rows, n_chunk, rows_hbm, idx_hbm, out_hbm,
                     idx_v, rows_v, sem):
    info = plsc.get_sparse_core_info()
    nw = info.num_cores * info.num_subcores
    b_per_w = n_rows // nw
    wid = lax.axis_index("s") * info.num_cores + lax.axis_index("c")
    base = wid * b_per_w

    def step(g, _):
        off = base + g * n_chunk
        pltpu.sync_copy(idx_hbm.at[pl.ds(off, n_chunk)], idx_v)
        pltpu.sync_copy(rows_hbm.at[pl.ds(off, n_chunk)], rows_v)
        pltpu.async_copy(rows_v, out_hbm.at[idx_v], sem).wait()
        return _

    lax.fori_loop(0, b_per_w // n_chunk, step, 0)


def _sc_scatter(rows, idx, n_chunk):
    n_rows, d = rows.shape
    mesh = plsc.VectorSubcoreMesh(core_axis_name="c", subcore_axis_name="s")
    kfn = functools.partial(
        pl.kernel,
        mesh=mesh,
        out_type=jax.ShapeDtypeStruct((n_rows, d), rows.dtype),
        scratch_types=[
            pltpu.VMEM((n_chunk,), jnp.int32),
            pltpu.VMEM((n_chunk, d), rows.dtype),
            pltpu.SemaphoreType.DMA,
        ],
    )(functools.partial(_sc_scatter_body, n_rows, n_chunk))
    return kfn(rows, idx)


def _rank_body(qaug_ref, qbrow_ref, bcol_ref, rank_ref, segs_ref):
    i = pl.program_id(0)

    @pl.when(i == 0)
    def _():
        bio16 = lax.broadcasted_iota(jnp.int32, (1, B), 1).astype(jnp.float32)
        onehot_nb = (bcol_ref[:, 0:1] == bio16).astype(jnp.float32)
        cnt = jnp.sum(onehot_nb, axis=0, keepdims=True)
        r16 = lax.broadcasted_iota(jnp.int32, (B, B), 0)
        c16 = lax.broadcasted_iota(jnp.int32, (B, B), 1)
        ltb = (r16 < c16).astype(jnp.float32)
        cum = jnp.dot(cnt, ltb, preferred_element_type=jnp.float32)
        pad = jnp.zeros((1, 128 - B), jnp.float32)
        cnt128 = jnp.concatenate([cnt, pad], axis=1)
        cum128 = jnp.concatenate([cum, pad], axis=1)
        rio = lax.broadcasted_iota(jnp.int32, (8, 128), 0)
        segs_ref[...] = jnp.where(rio == 0,
                                  jnp.broadcast_to(cnt128, (8, 128)),
                                  jnp.where(rio == 1,
                                            jnp.broadcast_to(cum128, (8, 128)),
                                            0.0))

    qb = qaug_ref[:, DIN + 3:DIN + 4]
    qrow = qbrow_ref[0:1, :]
    less = jnp.sum((qrow < qb).astype(jnp.float32), axis=1, keepdims=True)
    colidx = lax.broadcasted_iota(jnp.int32, (1, Q_PAD), 1)
    before = colidx < i * TQ
    eq_before = jnp.sum(
        jnp.where(jnp.logical_and(qrow == qb, before), 1.0, 0.0),
        axis=1, keepdims=True)
    rr = lax.broadcasted_iota(jnp.int32, (TQ, TQ), 0)
    cc = lax.broadcasted_iota(jnp.int32, (TQ, TQ), 1)
    ltq = (cc < rr).astype(jnp.float32)
    bio = lax.broadcasted_iota(jnp.int32, (1, B), 1).astype(jnp.float32)
    onehot = (qb == bio).astype(jnp.float32)
    pref = jnp.dot(ltq, onehot, preferred_element_type=jnp.float32)
    eq_tile = jnp.sum(onehot * pref, axis=1, keepdims=True)
    rank = less + eq_before + eq_tile
    rank_ref[...] = rank.astype(jnp.int32)


def _rank(qaug, qbrow, bcol):
    return pl.pallas_call(
        _rank_body,
        grid=(N_TILES,),
        in_specs=[
            pl.BlockSpec((TQ, AUGD), lambda i: (i, 0)),
            pl.BlockSpec((8, Q_PAD), lambda i: (0, 0)),
            pl.BlockSpec((N_PAD, 8), lambda i: (0, 0)),
        ],
        out_specs=[
            pl.BlockSpec((TQ, 1), lambda i: (i, 0)),
            pl.BlockSpec((8, 128), lambda i: (0, 0)),
        ],
        out_shape=[
            jax.ShapeDtypeStruct((Q_PAD, 1), jnp.int32),
            jax.ShapeDtypeStruct((8, 128), jnp.float32),
        ],
    )(qaug, qbrow, bcol)


def _knn_body(qaug_ref, xyzt_ref, brow_ref, segs_ref, idx_ref):
    qx = qaug_ref[:, DIN:DIN + 1]
    qy = qaug_ref[:, DIN + 1:DIN + 2]
    qz = qaug_ref[:, DIN + 2:DIN + 3]
    qb = qaug_ref[:, DIN + 3:DIN + 4]

    kcol = lax.broadcasted_iota(jnp.int32, (1, K), 1)
    brow = brow_ref[0:1, :]

    bmin = jnp.min(qb)
    bmax = jnp.max(qb)
    bio16 = lax.broadcasted_iota(jnp.int32, (1, B), 1).astype(jnp.float32)
    cnt = segs_ref[0:1, 0:B]
    cum = segs_ref[1:2, 0:B]
    inb = jnp.logical_and(bio16 >= bmin, bio16 <= bmax)
    minlen = jnp.min(jnp.where(inb, cnt, _BIGV))
    start = jnp.sum(jnp.where(bio16 == bmin, cum, 0.0)).astype(jnp.int32)
    end = jnp.sum(jnp.where(bio16 == bmax, cum + cnt, 0.0)).astype(jnp.int32)
    narrow = minlen >= jnp.float32(K)
    c0 = jnp.where(narrow, start // TN, 0)
    c1 = jnp.where(narrow, (end + TN - 1) // TN, N_CHUNKS)

    def chunk_body(c, carry):
        bestv, besti = carry
        off = c * TN
        sx = xyzt_ref[0:1, pl.ds(off, TN)]
        sy = xyzt_ref[1:2, pl.ds(off, TN)]
        sz = xyzt_ref[2:3, pl.ds(off, TN)]
        sb = brow_ref[0:1, pl.ds(off, TN)]
        dx = qx - sx
        dy = qy - sy
        dz = qz - sz
        d2 = dx * dx + dy * dy
        d2 = d2 + dz * dz
        d2 = d2 + jnp.where(qb != sb, 1e9, 0.0)
        gidx = (off + lax.broadcasted_iota(jnp.int32, (1, TN), 1)).astype(
            jnp.float32)
        wv = jnp.concatenate([bestv, d2], axis=1)
        wi = jnp.concatenate([besti, jnp.broadcast_to(gidx, (TQ, TN))], axis=1)

        bv = jnp.full((TQ, K), _BIGV, jnp.float32)
        bi = jnp.full((TQ, K), -1.0, jnp.float32)
        for k in range(K):
            m = jnp.min(wv, axis=1, keepdims=True)
            ji = jnp.min(jnp.where(wv == m, wi, _BIGI), axis=1, keepdims=True)
            wv = jnp.where(wi == ji, _BIGV, wv)
            sel = kcol == k
            bv = jnp.where(sel, m, bv)
            bi = jnp.where(sel, ji, bi)
        return bv, bi

    bestv0 = jnp.full((TQ, K), _BIGV, jnp.float32)
    besti0 = jnp.full((TQ, K), -1.0, jnp.float32)
    _, besti = lax.fori_loop(c0, c1, chunk_body, (bestv0, besti0))
    idx_ref[...] = besti.astype(jnp.int32)


def _knn(qaug, xyzt, brow, segs):
    nq = qaug.shape[0]
    return pl.pallas_call(
        _knn_body,
        grid=(nq // TQ,),
        in_specs=[
            pl.BlockSpec((TQ, AUGD), lambda i: (i, 0)),
            pl.BlockSpec((8, N_PAD), lambda i: (0, 0)),
            pl.BlockSpec((8, N_PAD), lambda i: (0, 0)),
            pl.BlockSpec((8, 128), lambda i: (0, 0)),
        ],
        out_specs=pl.BlockSpec((TQ, K), lambda i: (i, 0)),
        out_shape=jax.ShapeDtypeStruct((nq, K), jnp.int32),
    )(qaug, xyzt, brow, segs)


def _head_body(qaug_ref, xj_ref, lang_ref, wl1_ref, bl1_ref, bng_ref, bnb_ref,
               wl2_ref, bl2_ref, we_ref, be_ref, wv1_ref, bv1_ref, lng_ref,
               lnb_ref, wv2_ref, bv2_ref, out_ref):
    l = jnp.dot(lang_ref[...], wl1_ref[...],
                preferred_element_type=jnp.float32) + bl1_ref[...]
    l = l / jnp.sqrt(1.0 + 1e-5) * bng_ref[...] + bnb_ref[...]
    l = jnp.maximum(l, 0.0)
    l = jnp.dot(l, wl2_ref[...],
                preferred_element_type=jnp.float32) + bl2_ref[...]

    x_i = qaug_ref[:, 0:DIN]
    qb = qaug_ref[:, DIN + 3:DIN + 4]

    we = we_ref[...]
    be = be_ref[...]
    g = jnp.full((TQ, H), -_BIGV, jnp.float32)
    for k in range(K):
        xj = xj_ref[k]
        e = jnp.concatenate([x_i, xj - x_i], axis=1)
        h = jnp.dot(e, we, preferred_element_type=jnp.float32) + be
        g = jnp.maximum(g, jnp.maximum(h, 0.0))

    v = jnp.dot(g, wv1_ref[...],
                preferred_element_type=jnp.float32) + bv1_ref[...]
    mu = jnp.mean(v, axis=1, keepdims=True)
    var = jnp.mean((v - mu) * (v - mu), axis=1, keepdims=True)
    v = (v - mu) / jnp.sqrt(var + 1e-5) * lng_ref[...] + lnb_ref[...]
    v = jnp.maximum(v, 0.0)
    v = jnp.dot(v, wv2_ref[...],
                preferred_element_type=jnp.float32) + bv2_ref[...]

    bio = lax.broadcasted_iota(jnp.int32, (1, B), 1).astype(jnp.float32)
    onehot = (qb == bio).astype(jnp.float32)
    lq = jnp.dot(onehot, l, preferred_element_type=jnp.float32)

    num = jnp.sum(v * lq, axis=1, keepdims=True)
    nv = jnp.sqrt(jnp.sum(v * v, axis=1, keepdims=True))
    nl = jnp.sqrt(jnp.sum(lq * lq, axis=1, keepdims=True))
    den = jnp.maximum(nv * nl, 1e-8)
    out_ref[...] = jnp.broadcast_to(num / den, (TQ, 128))


def _head(qaug, xj3, lang, wl1, bl1, bng, bnb, wl2, bl2, we, be, wv1, bv1,
          lng, lnb, wv2, bv2):
    nq = qaug.shape[0]
    full = lambda shape: pl.BlockSpec(shape, lambda i: tuple(0 for _ in shape))
    return pl.pallas_call(
        _head_body,
        grid=(nq // TQ,),
        in_specs=[
            pl.BlockSpec((TQ, AUGD), lambda i: (i, 0)),
            pl.BlockSpec((K, TQ, DIN), lambda i: (0, i, 0)),
            full((B, L)),
            full((L, H)), full((1, H)), full((1, H)), full((1, H)),
            full((H, H)), full((1, H)),
            full((2 * DIN, H)), full((1, H)),
            full((H, H)), full((1, H)), full((1, H)), full((1, H)),
            full((H, H)), full((1, H)),
        ],
        out_specs=pl.BlockSpec((TQ, 128), lambda i: (i, 0)),
        out_shape=jax.ShapeDtypeStruct((nq, 128), jnp.float32),
    )(qaug, xj3, lang, wl1, bl1, bng, bnb, wl2, bl2, we, be, wv1, bv1,
      lng, lnb, wv2, bv2)


def kernel(support_xyz, batch_index, filtered_index, feats, lang_rel_feats,
           W_l1, b_l1, bn_g, bn_b, W_l2, b_l2,
           W_e, b_e, W_v1, b_v1, ln_g, ln_b, W_v2, b_v2):
    batch_index = batch_index.astype(jnp.int32)
    filtered_index = filtered_index.astype(jnp.int32)

    aug = jnp.concatenate([
        feats,
        support_xyz,
        batch_index[:, None].astype(jnp.float32),
        jnp.zeros((N, AUGD - DIN - 4), jnp.float32),
    ], axis=1)

    fi_pad = jnp.concatenate(
        [filtered_index, jnp.zeros((Q_PAD - Q,), jnp.int32)])

    qaug = _sc_gather(aug, fi_pad, 160)

    qbrow = jnp.broadcast_to(qaug[:, DIN + 3:DIN + 4].T, (8, Q_PAD))
    bcol = jnp.full((N_PAD, 8), 99.0, jnp.float32)
    bcol = bcol.at[0:N, 0].set(batch_index.astype(jnp.float32))
    rank, segs = _rank(qaug, qbrow, bcol)
    rank_flat = rank.reshape(-1)
    qaug_s = _sc_scatter(qaug, rank_flat, 80)

    xyzt = jnp.full((8, N_PAD), 1e5, jnp.float32)
    xyzt = xyzt.at[0:3, 0:N].set(support_xyz.T)
    brow = jnp.full((8, N_PAD), 99.0, jnp.float32)
    brow = brow.at[0, 0:N].set(batch_index.astype(jnp.float32))

    half = Q_PAD // 4
    scores_halves = []
    for hh in range(4):
        qh = qaug_s[hh * half:(hh + 1) * half]
        idx_h = _knn(qh, xyzt, brow, segs)
        idxf_h = idx_h.T.reshape(-1)
        xj_h = _sc_gather(feats, idxf_h, 320)
        xj3_h = xj_h.reshape(K, half, DIN)
        scores_halves.append(
            _head(qh, xj3_h, lang_rel_feats, W_l1, b_l1[None, :],
                  bn_g[None, :], bn_b[None, :], W_l2, b_l2[None, :],
                  W_e, b_e[None, :], W_v1, b_v1[None, :], ln_g[None, :],
                  ln_b[None, :], W_v2, b_v2[None, :]))
    scores_t = jnp.concatenate(scores_halves, axis=0)

    scores = _sc_gather(scores_t, rank_flat, 160)
    return scores[:Q, 0]
